# baseline (speedup 1.0000x reference)
import functools

import jax
import jax.numpy as jnp
from jax.experimental import pallas as pl
from jax.experimental.pallas import tpu as pltpu

_CG = 16
_EPS = 1e-3
_NS_ITERS = 10
_NB = 2


def _stats_kernel(x_ref, p_ref):
    j = pl.program_id(0)
    pp = None
    for k in range(_NB):
        x = x_ref[k]
        ones = jnp.ones((x.shape[0], 1), dtype=x.dtype)
        xa = jnp.concatenate([x, ones], axis=1)
        part = jax.lax.dot_general(
            x, xa, (((0,), (0,)), ((), ())),
            preferred_element_type=jnp.float32)
        pp = part if pp is None else pp + part

    @pl.when(j == 0)
    def _():
        p_ref[...] = pp

    @pl.when(j > 0)
    def _():
        p_ref[...] += pp


def _solve_kernel(p_ref, w_ref, b_ref, wf_ref, off_ref, *, inv_m):
    c = w_ref.shape[0]
    pt = p_ref[...]
    mean = pt[:, c:c + 1] * inv_m
    outer = jax.lax.dot_general(
        mean, mean, (((1,), (1,)), ((), ())),
        preferred_element_type=jnp.float32)
    rows = jax.lax.broadcasted_iota(jnp.int32, (c, c), 0)
    cols = jax.lax.broadcasted_iota(jnp.int32, (c, c), 1)
    blk = (rows // _CG) == (cols // _CG)
    maskf = jnp.where(blk, 1.0, 0.0).astype(jnp.float32)
    eyef = jnp.where(rows == cols, 1.0, 0.0).astype(jnp.float32)
    sigma = (pt[:, :c] * inv_m - outer) * maskf + _EPS * eyef

    rs = jnp.sum(sigma * sigma, axis=1, keepdims=True)
    f2 = jax.lax.dot_general(
        maskf, rs, (((1,), (0,)), ((), ())),
        preferred_element_type=jnp.float32)
    invf = jax.lax.rsqrt(f2)
    y = sigma * invf
    z = eyef
    dn = (((1,), (0,)), ((), ()))
    for _ in range(_NS_ITERS):
        t = 1.5 * eyef - 0.5 * jax.lax.dot_general(
            z, y, dn, preferred_element_type=jnp.float32)
        y = jax.lax.dot_general(y, t, dn, preferred_element_type=jnp.float32)
        z = jax.lax.dot_general(t, z, dn, preferred_element_type=jnp.float32)
    wm = z * jnp.sqrt(invf)
    wf = wm * w_ref[...]
    off = b_ref[...] - jax.lax.dot_general(
        mean, wf, (((0,), (1,)), ((), ())),
        preferred_element_type=jnp.float32)
    wf_ref[...] = wf
    off_ref[...] = off


def _apply_kernel(x_ref, wf_ref, off_ref, o_ref):
    wf = wf_ref[...]
    off = off_ref[...]
    dn = (((1,), (1,)), ((), ()))
    for k in range(_NB):
        o_ref[k] = jax.lax.dot_general(
            x_ref[k], wf, dn, preferred_element_type=jnp.float32) + off


def kernel(X, weight, bias):
    n, c, h, w = X.shape
    hw = h * w
    x2 = jnp.transpose(X, (0, 2, 3, 1)).reshape(n, hw, c)

    p2 = pl.pallas_call(
        _stats_kernel,
        grid=(n // _NB,),
        in_specs=[pl.BlockSpec((_NB, hw, c), lambda j: (j, 0, 0))],
        out_specs=pl.BlockSpec((c, c + 1), lambda j: (0, 0)),
        out_shape=jax.ShapeDtypeStruct((c, c + 1), jnp.float32),
        compiler_params=pltpu.CompilerParams(
            dimension_semantics=("arbitrary",),
            vmem_limit_bytes=56 * 1024 * 1024),
    )(x2)

    wf, off = pl.pallas_call(
        functools.partial(_solve_kernel, inv_m=1.0 / (n * hw)),
        out_shape=(jax.ShapeDtypeStruct((c, c), jnp.float32),
                   jax.ShapeDtypeStruct((1, c), jnp.float32)),
    )(p2, weight.reshape(c, 1), bias.reshape(1, c))

    y2 = pl.pallas_call(
        _apply_kernel,
        grid=(n // _NB,),
        in_specs=[pl.BlockSpec((_NB, hw, c), lambda j: (j, 0, 0)),
                  pl.BlockSpec((c, c), lambda j: (0, 0)),
                  pl.BlockSpec((1, c), lambda j: (0, 0))],
        out_specs=pl.BlockSpec((_NB, hw, c), lambda j: (j, 0, 0)),
        out_shape=jax.ShapeDtypeStruct((n, hw, c), jnp.float32),
        compiler_params=pltpu.CompilerParams(
            dimension_semantics=("arbitrary",),
            vmem_limit_bytes=56 * 1024 * 1024),
    )(x2, wf, off)

    return jnp.transpose(y2.reshape(n, h, w, c), (0, 3, 1, 2))


# speedup vs baseline: 1.0552x; 1.0552x over previous
import functools

import jax
import jax.numpy as jnp
from jax.experimental import pallas as pl
from jax.experimental.pallas import tpu as pltpu

_CG = 16
_EPS = 1e-3
_NS_ITERS = 10
_NB = 4
_NBS = 8


def _stats_kernel(x_ref, p_ref):
    j = pl.program_id(0)
    pp = None
    for k in range(_NBS):
        x = x_ref[k]
        ones = jnp.ones((x.shape[0], 1), dtype=x.dtype)
        xa = jnp.concatenate([x, ones], axis=1)
        part = jax.lax.dot_general(
            x, xa, (((0,), (0,)), ((), ())),
            preferred_element_type=jnp.float32)
        pp = part if pp is None else pp + part

    @pl.when(j == 0)
    def _():
        p_ref[...] = pp

    @pl.when(j > 0)
    def _():
        p_ref[...] += pp


def _solve_kernel(p_ref, w_ref, b_ref, wf_ref, off_ref, *, inv_m):
    c = w_ref.shape[0]
    pt = p_ref[...]
    mean = pt[:, c:c + 1] * inv_m
    outer = jax.lax.dot_general(
        mean, mean, (((1,), (1,)), ((), ())),
        preferred_element_type=jnp.float32)
    rows = jax.lax.broadcasted_iota(jnp.int32, (c, c), 0)
    cols = jax.lax.broadcasted_iota(jnp.int32, (c, c), 1)
    blk = (rows // _CG) == (cols // _CG)
    maskf = jnp.where(blk, 1.0, 0.0).astype(jnp.float32)
    eyef = jnp.where(rows == cols, 1.0, 0.0).astype(jnp.float32)
    sigma = (pt[:, :c] * inv_m - outer) * maskf + _EPS * eyef

    rs = jnp.sum(sigma * sigma, axis=1, keepdims=True)
    f2 = jax.lax.dot_general(
        maskf, rs, (((1,), (0,)), ((), ())),
        preferred_element_type=jnp.float32)
    invf = jax.lax.rsqrt(f2)
    y = sigma * invf
    z = eyef
    dn = (((1,), (0,)), ((), ()))
    for _ in range(_NS_ITERS):
        t = 1.5 * eyef - 0.5 * jax.lax.dot_general(
            z, y, dn, preferred_element_type=jnp.float32)
        y = jax.lax.dot_general(y, t, dn, preferred_element_type=jnp.float32)
        z = jax.lax.dot_general(t, z, dn, preferred_element_type=jnp.float32)
    wm = z * jnp.sqrt(invf)
    wf = wm * w_ref[...]
    off = b_ref[...] - jax.lax.dot_general(
        mean, wf, (((0,), (1,)), ((), ())),
        preferred_element_type=jnp.float32)
    wf_ref[...] = wf
    off_ref[...] = off


def _apply_kernel(x_ref, wf_ref, off_ref, o_ref):
    wf = wf_ref[...]
    off = off_ref[...]
    dn = (((1,), (1,)), ((), ()))
    for k in range(_NB):
        o_ref[k] = jax.lax.dot_general(
            x_ref[k], wf, dn, preferred_element_type=jnp.float32) + off


def kernel(X, weight, bias):
    n, c, h, w = X.shape
    hw = h * w
    x2 = jnp.transpose(X, (0, 2, 3, 1)).reshape(n, hw, c)

    p2 = pl.pallas_call(
        _stats_kernel,
        grid=(n // _NBS,),
        in_specs=[pl.BlockSpec((_NBS, hw, c), lambda j: (j, 0, 0))],
        out_specs=pl.BlockSpec((c, c + 1), lambda j: (0, 0)),
        out_shape=jax.ShapeDtypeStruct((c, c + 1), jnp.float32),
        compiler_params=pltpu.CompilerParams(
            dimension_semantics=("arbitrary",),
            vmem_limit_bytes=56 * 1024 * 1024),
    )(x2)

    wf, off = pl.pallas_call(
        functools.partial(_solve_kernel, inv_m=1.0 / (n * hw)),
        out_shape=(jax.ShapeDtypeStruct((c, c), jnp.float32),
                   jax.ShapeDtypeStruct((1, c), jnp.float32)),
    )(p2, weight.reshape(c, 1), bias.reshape(1, c))

    y2 = pl.pallas_call(
        _apply_kernel,
        grid=(n // _NB,),
        in_specs=[pl.BlockSpec((_NB, hw, c), lambda j: (j, 0, 0)),
                  pl.BlockSpec((c, c), lambda j: (0, 0)),
                  pl.BlockSpec((1, c), lambda j: (0, 0))],
        out_specs=pl.BlockSpec((_NB, hw, c), lambda j: (j, 0, 0)),
        out_shape=jax.ShapeDtypeStruct((n, hw, c), jnp.float32),
        compiler_params=pltpu.CompilerParams(
            dimension_semantics=("arbitrary",),
            vmem_limit_bytes=56 * 1024 * 1024),
    )(x2, wf, off)

    return jnp.transpose(y2.reshape(n, h, w, c), (0, 3, 1, 2))


# speedup vs baseline: 1.0763x; 1.0200x over previous
import functools

import jax
import jax.numpy as jnp
from jax.experimental import pallas as pl
from jax.experimental.pallas import tpu as pltpu

_CG = 16
_EPS = 1e-3
_NS_ITERS = 10
_NB = 4
_NSTEPS = 16


def _fused_kernel(x_ref, w_ref, b_ref, p_ref, o_ref, wf_s, off_s, *, inv_m):
    j = pl.program_id(0)
    c = w_ref.shape[0]

    @pl.when(j < _NSTEPS)
    def _():
        pp = None
        for k in range(_NB):
            x = x_ref[k]
            ones = jnp.ones((x.shape[0], 1), dtype=x.dtype)
            xa = jnp.concatenate([x, ones], axis=1)
            part = jax.lax.dot_general(
                x, xa, (((0,), (0,)), ((), ())),
                preferred_element_type=jnp.float32)
            pp = part if pp is None else pp + part

        @pl.when(j == 0)
        def _():
            p_ref[...] = pp

        @pl.when(j > 0)
        def _():
            p_ref[...] += pp

    @pl.when(j == _NSTEPS)
    def _():
        pt = p_ref[...]
        mean = pt[:, c:c + 1] * inv_m
        outer = jax.lax.dot_general(
            mean, mean, (((1,), (1,)), ((), ())),
            preferred_element_type=jnp.float32)
        rows = jax.lax.broadcasted_iota(jnp.int32, (c, c), 0)
        cols = jax.lax.broadcasted_iota(jnp.int32, (c, c), 1)
        blk = (rows // _CG) == (cols // _CG)
        maskf = jnp.where(blk, 1.0, 0.0).astype(jnp.float32)
        eyef = jnp.where(rows == cols, 1.0, 0.0).astype(jnp.float32)
        sigma = (pt[:, :c] * inv_m - outer) * maskf + _EPS * eyef
        rs = jnp.sum(sigma * sigma, axis=1, keepdims=True)
        f2 = jax.lax.dot_general(
            maskf, rs, (((1,), (0,)), ((), ())),
            preferred_element_type=jnp.float32)
        invf = jax.lax.rsqrt(f2)
        y = sigma * invf
        z = eyef
        dn = (((1,), (0,)), ((), ()))
        for _ in range(_NS_ITERS):
            t = 1.5 * eyef - 0.5 * jax.lax.dot_general(
                z, y, dn, preferred_element_type=jnp.float32)
            y = jax.lax.dot_general(y, t, dn, preferred_element_type=jnp.float32)
            z = jax.lax.dot_general(t, z, dn, preferred_element_type=jnp.float32)
        wm = z * jnp.sqrt(invf)
        wf = wm * w_ref[...]
        off = b_ref[...] - jax.lax.dot_general(
            mean, wf, (((0,), (1,)), ((), ())),
            preferred_element_type=jnp.float32)
        wf_s[...] = wf
        off_s[...] = off

    @pl.when(j > _NSTEPS)
    def _():
        wf = wf_s[...]
        off = off_s[...]
        dn = (((1,), (1,)), ((), ()))
        for k in range(_NB):
            o_ref[k] = jax.lax.dot_general(
                x_ref[k], wf, dn, preferred_element_type=jnp.float32) + off


def _x_index(j):
    return (jnp.where(j < _NSTEPS, jnp.minimum(j, _NSTEPS - 1),
                      jnp.maximum(j - _NSTEPS - 1, 0)), 0, 0)


def _o_index(j):
    return (jnp.maximum(j - _NSTEPS - 1, 0), 0, 0)


def kernel(X, weight, bias):
    n, c, h, w = X.shape
    hw = h * w
    x2 = jnp.transpose(X, (0, 2, 3, 1)).reshape(n, hw, c)
    nsteps = n // _NB
    assert nsteps == _NSTEPS

    p2, y2 = pl.pallas_call(
        functools.partial(_fused_kernel, inv_m=1.0 / (n * hw)),
        grid=(2 * nsteps + 1,),
        in_specs=[pl.BlockSpec((_NB, hw, c), _x_index),
                  pl.BlockSpec((c, 1), lambda j: (0, 0)),
                  pl.BlockSpec((1, c), lambda j: (0, 0))],
        out_specs=(pl.BlockSpec((c, c + 1), lambda j: (0, 0)),
                   pl.BlockSpec((_NB, hw, c), _o_index)),
        out_shape=(jax.ShapeDtypeStruct((c, c + 1), jnp.float32),
                   jax.ShapeDtypeStruct((n, hw, c), jnp.float32)),
        scratch_shapes=[
            pltpu.VMEM((c, c), jnp.float32),
            pltpu.VMEM((1, c), jnp.float32),
        ],
        compiler_params=pltpu.CompilerParams(
            dimension_semantics=("arbitrary",),
            vmem_limit_bytes=56 * 1024 * 1024),
    )(x2, weight.reshape(c, 1), bias.reshape(1, c))

    return jnp.transpose(y2.reshape(n, h, w, c), (0, 3, 1, 2))
